# baseline (device time: 86607 ns/iter reference)
import jax
import jax.numpy as jnp
from jax import lax
from jax.experimental import pallas as pl
from jax.experimental.pallas import tpu as pltpu

N_DEV = 16
SQ = 256
SKV = 4096
H_PER = 8
DH = 128
D_MODEL = 1024
SCALE = 0.08838834764831843
BLK = 64

_R = [list(range(r, 64, 3)) for r in range(3)]
_ORDER = _R[0] + _R[1] + _R[2]
_POS = {kb: i * BLK for i, kb in enumerate(_ORDER)}
_CLS = {r: (_POS[_R[r][0]], _POS[_R[r][-1]] + BLK) for r in range(3)}
_MAIN = {}
_EXTRA = {}
for _qb in range(4):
    _r = (-_qb) % 3
    _MAIN[_qb] = _CLS[_r]
    _EXTRA[_qb] = [(_POS[kb], _POS[kb] + BLK)
                   for kb in dict.fromkeys((0, _qb)) if kb % 3 != _r]


def _body(x_ref, wq_hbm, k_hbm, v_hbm, wo_hbm, out_ref,
          ctx_ref, acc_ref, a_land, b_land, kbuf, vbuf,
          wq_ref, wo_ref, ksems, vsems, wsems,
          send_sems, recv_sems):
    my = lax.axis_index("i")
    plane = my // 4
    pos = lax.rem(my, 4)
    p_right = plane * 4 + lax.rem(pos + 1, 4)
    p_left = plane * 4 + lax.rem(pos + 3, 4)
    z_next = lax.rem(plane + 1, 4) * 4 + pos
    z_prev = lax.rem(plane + 3, 4) * 4 + pos

    barrier_sem = pltpu.get_barrier_semaphore()
    for nbr in (p_left, p_right, z_prev, z_next):
        pl.semaphore_signal(
            barrier_sem, inc=1,
            device_id=(nbr,), device_id_type=pl.DeviceIdType.MESH,
        )
    pl.semaphore_wait(barrier_sem, 4)

    def kv_copies(h, slot):
        cps = []
        for i, kb in enumerate(_ORDER):
            cps.append(pltpu.make_async_copy(
                k_hbm.at[pl.ds(kb * BLK, BLK), h, :],
                kbuf.at[slot, pl.ds(i * BLK, BLK)], ksems.at[slot]))
            cps.append(pltpu.make_async_copy(
                v_hbm.at[pl.ds(kb * BLK, BLK), h, :],
                vbuf.at[slot, pl.ds(i * BLK, BLK)], vsems.at[slot]))
        return cps

    wq_cp = pltpu.make_async_copy(
        wq_hbm.at[:, pl.ds(my * H_PER * DH, H_PER * DH)], wq_ref, wsems.at[0])
    wo_cp = pltpu.make_async_copy(
        wo_hbm.at[pl.ds(my * H_PER * DH, H_PER * DH), :], wo_ref, wsems.at[1])
    wq_cp.start()
    wo_cp.start()

    for c in kv_copies(0, 0):
        c.start()

    bf16 = jnp.bfloat16
    wq_cp.wait()
    q = jnp.dot(x_ref[:, :].astype(bf16), wq_ref[:, :].astype(bf16),
                preferred_element_type=jnp.float32)

    for h in range(H_PER):
        slot = h % 2
        if h + 1 < H_PER:
            for c in kv_copies(h + 1, (h + 1) % 2):
                c.start()
        for c in kv_copies(h, slot):
            c.wait()
        kbf = kbuf[slot].astype(bf16)
        vbf = vbuf[slot].astype(bf16)
        qh = q[:, h * DH:(h + 1) * DH].astype(bf16)
        for qb in range(4):
            q_qb = qh[qb * BLK:(qb + 1) * BLK]
            spans = [_MAIN[qb]] + _EXTRA[qb]
            pieces = [
                lax.dot_general(
                    q_qb, kbf[lo:hi], (((1,), (1,)), ((), ())),
                    preferred_element_type=jnp.float32,
                ) * SCALE
                for lo, hi in spans
            ]
            m = pieces[0].max(axis=1, keepdims=True)
            for p in pieces[1:]:
                m = jnp.maximum(m, p.max(axis=1, keepdims=True))
            es = [jnp.exp(p - m) for p in pieces]
            ssum = es[0].sum(axis=1, keepdims=True)
            for e in es[1:]:
                ssum = ssum + e.sum(axis=1, keepdims=True)
            ctx_qb = None
            for e, (lo, hi) in zip(es, spans):
                w = (e / ssum).astype(bf16)
                d = jnp.dot(w, vbf[lo:hi],
                            preferred_element_type=jnp.float32)
                ctx_qb = d if ctx_qb is None else ctx_qb + d
            ctx_ref[qb * BLK:(qb + 1) * BLK,
                    h * DH:(h + 1) * DH] = ctx_qb

    wo_cp.wait()
    acc_ref[:, :] = jnp.dot(ctx_ref[:, :].astype(bf16),
                            wo_ref[:, :].astype(bf16),
                            preferred_element_type=jnp.float32)


    def hop(sem_idx, src, dst, target):
        rdma = pltpu.make_async_remote_copy(
            src_ref=src, dst_ref=dst,
            send_sem=send_sems.at[sem_idx],
            recv_sem=recv_sems.at[sem_idx],
            device_id=(target,),
            device_id_type=pl.DeviceIdType.MESH,
        )
        rdma.start()
        rdma.wait()

    for s in range(3):
        c_send = lax.rem(pos - s + 4, 4)
        c_recv = lax.rem(pos - s + 3, 4)
        hop(s,
            acc_ref.at[pl.ds(c_send * 64, 64), :],
            a_land.at[s],
            p_right)
        acc_ref[pl.ds(c_recv * 64, 64), :] += a_land[s]

    q_own = lax.rem(pos + 1, 4)
    row0 = q_own * 64

    for s in range(3):
        t_send = lax.rem(plane - s + 4, 4)
        t_recv = lax.rem(plane - s + 3, 4)
        hop(3 + s,
            acc_ref.at[pl.ds(row0 + t_send * 16, 16), :],
            b_land.at[s],
            z_next)
        acc_ref[pl.ds(row0 + t_recv * 16, 16), :] += b_land[s]
    for s in range(3):
        t_send = lax.rem(plane + 1 - s + 4, 4)
        hop(6 + s,
            acc_ref.at[pl.ds(row0 + t_send * 16, 16), :],
            acc_ref.at[pl.ds(row0 + t_send * 16, 16), :],
            z_next)

    for s in range(3):
        c_send = lax.rem(pos + 1 - s + 4, 4)
        hop(9 + s,
            acc_ref.at[pl.ds(c_send * 64, 64), :],
            acc_ref.at[pl.ds(c_send * 64, 64), :],
            p_right)

    out_ref[:, :] = acc_ref[:, :]


def kernel(x, Wq, K_ext, V_ext, Wo):
    x2 = x.reshape(SQ, D_MODEL)
    k2 = K_ext.reshape(SKV, H_PER, DH)
    v2 = V_ext.reshape(SKV, H_PER, DH)

    out = pl.pallas_call(
        _body,
        out_shape=jax.ShapeDtypeStruct((SQ, D_MODEL), jnp.float32),
        in_specs=[
            pl.BlockSpec(memory_space=pltpu.VMEM),
            pl.BlockSpec(memory_space=pl.ANY),
            pl.BlockSpec(memory_space=pl.ANY),
            pl.BlockSpec(memory_space=pl.ANY),
            pl.BlockSpec(memory_space=pl.ANY),
        ],
        out_specs=pl.BlockSpec(memory_space=pltpu.VMEM),
        scratch_shapes=[
            pltpu.VMEM((SQ, H_PER * DH), jnp.float32),
            pltpu.VMEM((SQ, D_MODEL), jnp.float32),
            pltpu.VMEM((3, 64, D_MODEL), jnp.float32),
            pltpu.VMEM((3, 16, D_MODEL), jnp.float32),
            pltpu.VMEM((2, SKV, DH), jnp.float32),
            pltpu.VMEM((2, SKV, DH), jnp.float32),
            pltpu.VMEM((D_MODEL, H_PER * DH), jnp.float32),
            pltpu.VMEM((H_PER * DH, D_MODEL), jnp.float32),
            pltpu.SemaphoreType.DMA((2,)),
            pltpu.SemaphoreType.DMA((2,)),
            pltpu.SemaphoreType.DMA((2,)),
            pltpu.SemaphoreType.DMA((12,)),
            pltpu.SemaphoreType.DMA((12,)),
        ],
        compiler_params=pltpu.CompilerParams(
            collective_id=0, vmem_limit_bytes=60 * 1024 * 1024),
    )(x2, Wq, k2, v2, Wo)
    return out.reshape(1, SQ, D_MODEL)


# device time: 63355 ns/iter; 1.3670x vs baseline; 1.3670x over previous
import jax
import jax.numpy as jnp
from jax import lax
from jax.experimental import pallas as pl
from jax.experimental.pallas import tpu as pltpu

N_DEV = 16
SQ = 256
SKV = 4096
H_PER = 8
DH = 128
D_MODEL = 1024
SCALE = 0.08838834764831843
BLK = 64


def _body(x_ref, wq_hbm, k_hbm, v_hbm, wo_hbm, out_ref,
          ctx_ref, acc_ref, a_land, a_stage, bh_land, bq_land, cbuf, kbuf, vbuf,
          wq_ref, wo_ref, ksems, vsems, wsems,
          send_sems, recv_sems):
    my = lax.axis_index("i")
    plane = my // 4
    pos = lax.rem(my, 4)
    p_right = plane * 4 + lax.rem(pos + 1, 4)
    p_left = plane * 4 + lax.rem(pos + 3, 4)

    barrier_sem = pltpu.get_barrier_semaphore()
    for nbr in (p_left, p_right, (plane ^ 1) * 4 + pos, (plane ^ 2) * 4 + pos):
        pl.semaphore_signal(
            barrier_sem, inc=1,
            device_id=(nbr,), device_id_type=pl.DeviceIdType.MESH,
        )
    pl.semaphore_wait(barrier_sem, 4)

    def kv_copies(h, slot):
        kc = pltpu.make_async_copy(
            k_hbm.at[:, h, :], kbuf.at[slot], ksems.at[slot])
        vc = pltpu.make_async_copy(
            v_hbm.at[:, h, :], vbuf.at[slot], vsems.at[slot])
        return kc, vc

    wq_cp = pltpu.make_async_copy(
        wq_hbm.at[:, pl.ds(my * H_PER * DH, H_PER * DH)], wq_ref, wsems.at[0])
    wo_cp = pltpu.make_async_copy(
        wo_hbm.at[pl.ds(my * H_PER * DH, H_PER * DH), :], wo_ref, wsems.at[1])
    wq_cp.start()
    wo_cp.start()

    kc0, vc0 = kv_copies(0, 0)
    kc0.start()
    vc0.start()

    bf16 = jnp.bfloat16
    wq_cp.wait()
    q = jnp.dot(x_ref[:, :].astype(bf16), wq_ref[:, :].astype(bf16),
                preferred_element_type=jnp.float32) * SCALE

    row = lax.broadcasted_iota(jnp.int32, (SQ, SKV), 0)
    col = lax.broadcasted_iota(jnp.int32, (SQ, SKV), 1)
    qb = row // BLK
    kb = col // BLK
    mask = (qb == kb) | (kb == 0) | (((qb + kb) % 3) == 0)

    for h in range(H_PER):
        slot = h % 2
        if h + 1 < H_PER:
            kc, vc = kv_copies(h + 1, (h + 1) % 2)
            kc.start()
            vc.start()
        kw, vw = kv_copies(h, slot)
        kw.wait()
        vw.wait()
        qh = q[:, h * DH:(h + 1) * DH].astype(bf16)
        scores = lax.dot_general(
            qh, kbuf[slot].astype(bf16), (((1,), (1,)), ((), ())),
            preferred_element_type=jnp.float32,
        )
        e = jnp.where(mask, jnp.exp(scores), 0.0)
        s = jnp.sum(e, axis=1, keepdims=True)
        ctx_ref[:, h * DH:(h + 1) * DH] = jnp.dot(
            e.astype(bf16), vbuf[slot].astype(bf16),
            preferred_element_type=jnp.float32) / s


    def hop(sem_idx, src, dst, target):
        rdma = pltpu.make_async_remote_copy(
            src_ref=src, dst_ref=dst,
            send_sem=send_sems.at[sem_idx],
            recv_sem=recv_sems.at[sem_idx],
            device_id=(target,),
            device_id_type=pl.DeviceIdType.MESH,
        )
        rdma.start()
        rdma.wait()

    def a_rdma(s):
        return pltpu.make_async_remote_copy(
            src_ref=a_stage.at[s], dst_ref=a_land.at[s],
            send_sem=send_sems.at[s], recv_sem=recv_sems.at[s],
            device_id=(p_right,), device_id_type=pl.DeviceIdType.MESH,
        )

    wo_cp.wait()
    wo_bf = wo_ref[:, :].astype(bf16)
    for s in range(4):
        c = lax.rem(pos - s + 4, 4)
        chunk = lax.dot_general(
            ctx_ref[pl.ds(c * 64, 64), :].astype(bf16),
            wo_bf, (((1,), (0,)), ((), ())),
            preferred_element_type=jnp.float32,
        )
        if s > 0:
            a_rdma(s - 1).wait()
            chunk = chunk + a_land[s - 1].astype(jnp.float32)
        acc_ref[pl.ds(c * 64, 64), :] = chunk
        if s < 3:
            a_stage[s] = chunk.astype(bf16)
            a_rdma(s).start()

    q_own = lax.rem(pos + 1, 4)
    row0 = q_own * 64

    z_far = (plane ^ 2) * 4 + pos
    z_adj = (plane ^ 1) * 4 + pos
    half_own = plane // 2
    hop(3,
        acc_ref.at[pl.ds(row0 + (1 - half_own) * 32, 32), :],
        bh_land,
        z_far)
    acc_ref[pl.ds(row0 + half_own * 32, 32), :] += bh_land[:, :]
    hop(4,
        acc_ref.at[pl.ds(row0 + (plane ^ 1) * 16, 16), :],
        bq_land,
        z_adj)
    acc_ref[pl.ds(row0 + plane * 16, 16), :] += bq_land[:, :]
    def rdma_desc(sem_idx, src, dst, target):
        return pltpu.make_async_remote_copy(
            src_ref=src, dst_ref=dst,
            send_sem=send_sems.at[sem_idx],
            recv_sem=recv_sems.at[sem_idx],
            device_id=(target,),
            device_id_type=pl.DeviceIdType.MESH,
        )

    def c_hop(base, s, qoff, qlen):
        c_s = lax.rem(pos + 1 - s + 4, 4)
        rows = pl.ds(c_s * 64 + qoff, qlen)
        return rdma_desc(base + s, cbuf.at[rows, :], cbuf.at[rows, :],
                         p_right)

    def stage(qoff, qlen):
        cbuf[pl.ds(row0 + qoff, qlen), :] = (
            acc_ref[pl.ds(row0 + qoff, qlen), :].astype(bf16))

    qoff0 = plane * 16
    qoff1 = (plane ^ 1) * 16
    qoff23 = (1 - half_own) * 32
    r_zq = pl.ds(row0 + plane * 16, 16)
    r_zh = pl.ds(row0 + half_own * 32, 32)

    stage(qoff0, 16)
    c00 = c_hop(7, 0, qoff0, 16)
    c00.start()
    b5 = rdma_desc(5, acc_ref.at[r_zq, :], acc_ref.at[r_zq, :], z_adj)
    b5.start()
    c00.wait()
    c01 = c_hop(7, 1, qoff0, 16)
    c01.start()
    b5.wait()
    stage(qoff1, 16)
    c10 = c_hop(10, 0, qoff1, 16)
    c10.start()
    b6 = rdma_desc(6, acc_ref.at[r_zh, :], acc_ref.at[r_zh, :], z_far)
    b6.start()
    c01.wait()
    c02 = c_hop(7, 2, qoff0, 16)
    c02.start()
    c10.wait()
    c11 = c_hop(10, 1, qoff1, 16)
    c11.start()
    b6.wait()
    stage(qoff23, 32)
    c20 = c_hop(13, 0, qoff23, 32)
    c20.start()
    c02.wait()
    c11.wait()
    c12 = c_hop(10, 2, qoff1, 16)
    c12.start()
    c20.wait()
    c21 = c_hop(13, 1, qoff23, 32)
    c21.start()
    c12.wait()
    c21.wait()
    c22 = c_hop(13, 2, qoff23, 32)
    c22.start()
    c22.wait()

    out_ref[:, :] = cbuf[:, :].astype(jnp.float32)
    out_ref[pl.ds(row0, 64), :] = acc_ref[pl.ds(row0, 64), :]


def kernel(x, Wq, K_ext, V_ext, Wo):
    x2 = x.reshape(SQ, D_MODEL)
    k2 = K_ext.reshape(SKV, H_PER, DH)
    v2 = V_ext.reshape(SKV, H_PER, DH)

    out = pl.pallas_call(
        _body,
        out_shape=jax.ShapeDtypeStruct((SQ, D_MODEL), jnp.float32),
        in_specs=[
            pl.BlockSpec(memory_space=pltpu.VMEM),
            pl.BlockSpec(memory_space=pl.ANY),
            pl.BlockSpec(memory_space=pl.ANY),
            pl.BlockSpec(memory_space=pl.ANY),
            pl.BlockSpec(memory_space=pl.ANY),
        ],
        out_specs=pl.BlockSpec(memory_space=pltpu.VMEM),
        scratch_shapes=[
            pltpu.VMEM((SQ, H_PER * DH), jnp.float32),
            pltpu.VMEM((SQ, D_MODEL), jnp.float32),
            pltpu.VMEM((3, 64, D_MODEL), jnp.bfloat16),
            pltpu.VMEM((3, 64, D_MODEL), jnp.bfloat16),
            pltpu.VMEM((32, D_MODEL), jnp.float32),
            pltpu.VMEM((16, D_MODEL), jnp.float32),
            pltpu.VMEM((SQ, D_MODEL), jnp.bfloat16),
            pltpu.VMEM((2, SKV, DH), jnp.float32),
            pltpu.VMEM((2, SKV, DH), jnp.float32),
            pltpu.VMEM((D_MODEL, H_PER * DH), jnp.float32),
            pltpu.VMEM((H_PER * DH, D_MODEL), jnp.float32),
            pltpu.SemaphoreType.DMA((2,)),
            pltpu.SemaphoreType.DMA((2,)),
            pltpu.SemaphoreType.DMA((2,)),
            pltpu.SemaphoreType.DMA((16,)),
            pltpu.SemaphoreType.DMA((16,)),
        ],
        compiler_params=pltpu.CompilerParams(
            collective_id=0, vmem_limit_bytes=60 * 1024 * 1024),
    )(x2, Wq, k2, v2, Wo)
    return out.reshape(1, SQ, D_MODEL)
